# Initial kernel scaffold; baseline (speedup 1.0000x reference)
import jax
import jax.numpy as jnp
from jax.experimental import pallas as pl


def kernel(dh_outputs, dh_hidden, global_pointer, batch_size, story, domain, context_len, kb_len, conv_len, memory_mask, memory_story, W1, b1, W2, b2, C_know):
    raise NotImplementedError("write your pallas kernel here")


# speedup vs baseline: 1.0010x; 1.0010x over previous
import functools

import jax
import jax.numpy as jnp
from jax import lax
from jax.experimental import pallas as pl
from jax.experimental.pallas import tpu as pltpu
from jax.experimental.pallas import tpu_sc as plsc

B, S, D, M, V = 64, 50, 128, 2048, 100000
TOPK = 12
TV = 2000
NW = 32
ROWS_PER_W = B // NW
CHUNK = 128
LANES = 16


def _ivec_body(dh_ref, h_ref, w1_ref, b1_ref, w2_ref, b2_ref, out_ref):
    x = dh_ref[...]
    h = h_ref[0]
    hb = jnp.broadcast_to(h[:, None, :], (B, S, D))
    cat = jnp.concatenate([hb, x], axis=2).reshape(B * S, 2 * D)
    t = jnp.tanh(jnp.dot(cat, w1_ref[...],
                         preferred_element_type=jnp.float32) + b1_ref[...])
    q = (jnp.dot(t, w2_ref[...],
                 preferred_element_type=jnp.float32) + b2_ref[...])
    q = q.reshape(B, S, D)
    q = q - jnp.max(q, axis=1, keepdims=True)
    e = jnp.exp(q)
    q = e / jnp.sum(e, axis=1, keepdims=True)
    out_ref[...] = jnp.sum(q * x, axis=1)


def _scores_body(c_ref, iv_ref, out_ref):
    out_ref[...] = lax.dot_general(
        c_ref[...], iv_ref[...], (((1,), (1,)), ((), ())),
        preferred_element_type=jnp.float32)


def _topk_body(l_ref, out_ref):
    l = l_ref[...]
    pos = lax.broadcasted_iota(jnp.int32, (B, M), 1)
    cols = []
    for _ in range(TOPK):
        v = jnp.max(l, axis=1, keepdims=True)
        idx = jnp.min(jnp.where(l == v, pos, M), axis=1, keepdims=True)
        cols.append(idx)
        l = jnp.where(pos == idx, -jnp.inf, l)
    out_ref[...] = jnp.concatenate(cols, axis=1)


def _sc_logits(scores_flat, story, gp, kb_len, ctx_len):
    mesh = plsc.VectorSubcoreMesh(core_axis_name="c", subcore_axis_name="s")

    @functools.partial(
        pl.kernel, mesh=mesh,
        out_type=jax.ShapeDtypeStruct((B, M), jnp.float32),
        scratch_types=[
            pltpu.VMEM((M,), jnp.int32),
            pltpu.VMEM((M,), jnp.int32),
            pltpu.VMEM((M,), jnp.float32),
            pltpu.VMEM((M,), jnp.float32),
            pltpu.VMEM((B, LANES), jnp.int32),
            pltpu.VMEM((B, LANES), jnp.int32),
            pltpu.SemaphoreType.DMA,
        ],
    )
    def k(scores_hbm, story_hbm, gp_hbm, kb_hbm, ctx_hbm, out_hbm,
          story_v, idx_v, sc_v, gpv, kb_v, ctx_v, sem):
        cid = lax.axis_index("c")
        sid = lax.axis_index("s")
        w = sid * 2 + cid
        pltpu.sync_copy(kb_hbm, kb_v)
        pltpu.sync_copy(ctx_hbm, ctx_v)
        for r in range(ROWS_PER_W):
            b = w * ROWS_PER_W + r
            pltpu.sync_copy(story_hbm.at[b], story_v)
            pltpu.sync_copy(gp_hbm.at[b], gpv)

            def build(j, carry):
                s16 = story_v[pl.ds(j * LANES, LANES)]
                idx_v[pl.ds(j * LANES, LANES)] = s16 * B + b
                return carry
            lax.fori_loop(0, M // LANES, build, 0)

            copies = [
                pltpu.async_copy(
                    scores_hbm.at[idx_v.at[pl.ds(j * CHUNK, CHUNK)]],
                    sc_v.at[pl.ds(j * CHUNK, CHUNK)], sem)
                for j in range(M // CHUNK)
            ]
            for cp in copies:
                cp.wait()

            kb16 = kb_v[b]
            ctx16 = ctx_v[b]


            def comp(j, carry):
                posv = j * LANES + lax.iota(jnp.int32, LANES)
                sraw = sc_v[pl.ds(j * LANES, LANES)] * gpv[pl.ds(j * LANES, LANES)]
                badm = ((posv >= kb16) & (posv < ctx16 - 1)) | (posv >= ctx16)
                xm = jnp.where(badm, jnp.float32(-1e9), sraw)
                gpv[pl.ds(j * LANES, LANES)] = 1.0 / (1.0 + jnp.exp(-xm))
                return carry
            lax.fori_loop(0, M // LANES, comp, 0)
            pltpu.sync_copy(gpv, out_hbm.at[b])

    return k(scores_flat, story, gp, kb_len, ctx_len)


def kernel(dh_outputs, dh_hidden, global_pointer, batch_size, story, domain,
           context_len, kb_len, conv_len, memory_mask, memory_story,
           W1, b1, W2, b2, C_know):
    i_vec = pl.pallas_call(
        _ivec_body,
        out_shape=jax.ShapeDtypeStruct((B, D), jnp.float32),
    )(dh_outputs, dh_hidden, W1, b1.reshape(1, D), W2, b2.reshape(1, D))

    scoresT = pl.pallas_call(
        _scores_body,
        grid=(V // TV,),
        in_specs=[pl.BlockSpec((TV, D), lambda i: (i, 0)),
                  pl.BlockSpec((B, D), lambda i: (0, 0))],
        out_specs=pl.BlockSpec((TV, B), lambda i: (i, 0)),
        out_shape=jax.ShapeDtypeStruct((V, B), jnp.float32),
    )(C_know, i_vec)

    kb_b = jnp.broadcast_to(kb_len.astype(jnp.int32)[:, None], (B, LANES))
    ctx_b = jnp.broadcast_to(context_len.astype(jnp.int32)[:, None], (B, LANES))
    logits = _sc_logits(scoresT.reshape(V * B), story, global_pointer, kb_b, ctx_b)

    toppi = pl.pallas_call(
        _topk_body,
        out_shape=jax.ShapeDtypeStruct((B, TOPK), jnp.int32),
    )(logits)
    return toppi, i_vec


# speedup vs baseline: 1.2596x; 1.2583x over previous
import functools

import jax
import jax.numpy as jnp
from jax import lax
from jax.experimental import pallas as pl
from jax.experimental.pallas import tpu as pltpu
from jax.experimental.pallas import tpu_sc as plsc

B, S, D, M, V = 64, 50, 128, 2048, 100000
TOPK = 12
TV = 2000
NW = 32
ROWS_PER_W = B // NW
CHUNK = 128
LANES = 16


def _ivec_body(dh_ref, h_ref, w1_ref, b1_ref, w2_ref, b2_ref, out_ref):
    x = dh_ref[...]
    h = h_ref[0]
    hb = jnp.broadcast_to(h[:, None, :], (B, S, D))
    cat = jnp.concatenate([hb, x], axis=2).reshape(B * S, 2 * D)
    t = jnp.tanh(jnp.dot(cat, w1_ref[...],
                         preferred_element_type=jnp.float32) + b1_ref[...])
    q = (jnp.dot(t, w2_ref[...],
                 preferred_element_type=jnp.float32) + b2_ref[...])
    q = q.reshape(B, S, D)
    q = q - jnp.max(q, axis=1, keepdims=True)
    e = jnp.exp(q)
    q = e / jnp.sum(e, axis=1, keepdims=True)
    out_ref[...] = jnp.sum(q * x, axis=1)


def _scores_body(c_ref, iv_ref, out_ref):
    iv = iv_ref[...]
    r = lax.dot_general(c_ref[...], iv, (((1,), (1,)), ((), ())),
                        preferred_element_type=jnp.float32)
    r3 = r.reshape(TV // 2, 2, B)
    out_ref[...] = jnp.concatenate([r3[:, 0, :], r3[:, 1, :]], axis=1)


def _topk_body(l_ref, out_ref):
    l = l_ref[...]
    pos = lax.broadcasted_iota(jnp.int32, (B, M), 1)
    cols = []
    for _ in range(TOPK):
        v = jnp.max(l, axis=1, keepdims=True)
        idx = jnp.min(jnp.where(l == v, pos, M), axis=1, keepdims=True)
        cols.append(idx)
        l = jnp.where(pos == idx, -jnp.inf, l)
    out_ref[...] = jnp.concatenate(cols, axis=1)


def _sc_logits(scores_flat, story, gp, kb_len, ctx_len):
    mesh = plsc.VectorSubcoreMesh(core_axis_name="c", subcore_axis_name="s")

    @functools.partial(
        pl.kernel, mesh=mesh,
        out_type=jax.ShapeDtypeStruct((B, M), jnp.float32),
        scratch_types=[
            pltpu.VMEM((M,), jnp.int32),
            pltpu.VMEM((M,), jnp.int32),
            pltpu.VMEM((M,), jnp.float32),
            pltpu.VMEM((M,), jnp.float32),
            pltpu.VMEM((B, LANES), jnp.int32),
            pltpu.VMEM((B, LANES), jnp.int32),
            pltpu.SemaphoreType.DMA,
        ],
    )
    def k(scores_hbm, story_hbm, gp_hbm, kb_hbm, ctx_hbm, out_hbm,
          story_v, idx_v, sc_v, gpv, kb_v, ctx_v, sem):
        cid = lax.axis_index("c")
        sid = lax.axis_index("s")
        w = sid * 2 + cid
        pltpu.sync_copy(kb_hbm, kb_v)
        pltpu.sync_copy(ctx_hbm, ctx_v)
        for r in range(ROWS_PER_W):
            b = w * ROWS_PER_W + r
            pltpu.sync_copy(story_hbm.at[b], story_v)
            pltpu.sync_copy(gp_hbm.at[b], gpv)

            def build(j, carry):
                s16 = story_v[pl.ds(j * LANES, LANES)]
                idx_v[pl.ds(j * LANES, LANES)] = s16 * B + b
                return carry
            lax.fori_loop(0, M // LANES, build, 0)

            copies = [
                pltpu.async_copy(
                    scores_hbm.at[idx_v.at[pl.ds(j * CHUNK, CHUNK)]],
                    sc_v.at[pl.ds(j * CHUNK, CHUNK)], sem)
                for j in range(M // CHUNK)
            ]
            for cp in copies:
                cp.wait()

            kb16 = kb_v[b]
            ctx16 = ctx_v[b]


            def comp(j, carry):
                posv = j * LANES + lax.iota(jnp.int32, LANES)
                sraw = sc_v[pl.ds(j * LANES, LANES)] * gpv[pl.ds(j * LANES, LANES)]
                badm = ((posv >= kb16) & (posv < ctx16 - 1)) | (posv >= ctx16)
                xm = jnp.where(badm, jnp.float32(-1e9), sraw)
                gpv[pl.ds(j * LANES, LANES)] = 1.0 / (1.0 + jnp.exp(-xm))
                return carry
            lax.fori_loop(0, M // LANES, comp, 0)
            pltpu.sync_copy(gpv, out_hbm.at[b])

    return k(scores_flat, story, gp, kb_len, ctx_len)


def kernel(dh_outputs, dh_hidden, global_pointer, batch_size, story, domain,
           context_len, kb_len, conv_len, memory_mask, memory_story,
           W1, b1, W2, b2, C_know):
    i_vec = pl.pallas_call(
        _ivec_body,
        out_shape=jax.ShapeDtypeStruct((B, D), jnp.float32),
    )(dh_outputs, dh_hidden, W1, b1.reshape(1, D), W2, b2.reshape(1, D))

    scores2d = pl.pallas_call(
        _scores_body,
        grid=(V // TV,),
        in_specs=[pl.BlockSpec((TV, D), lambda i: (i, 0)),
                  pl.BlockSpec((B, D), lambda i: (0, 0))],
        out_specs=pl.BlockSpec((TV // 2, 2 * B), lambda i: (i, 0)),
        out_shape=jax.ShapeDtypeStruct((V // 2, 2 * B), jnp.float32),
    )(C_know, i_vec)
    scores_flat = scores2d.reshape(V * B)

    kb_b = jnp.broadcast_to(kb_len.astype(jnp.int32)[:, None], (B, LANES))
    ctx_b = jnp.broadcast_to(context_len.astype(jnp.int32)[:, None], (B, LANES))
    logits = _sc_logits(scores_flat, story, global_pointer, kb_b, ctx_b)

    toppi = pl.pallas_call(
        _topk_body,
        out_shape=jax.ShapeDtypeStruct((B, TOPK), jnp.int32),
    )(logits)
    return toppi, i_vec


# speedup vs baseline: 1.6605x; 1.3183x over previous
import functools

import jax
import jax.numpy as jnp
from jax import lax
from jax.experimental import pallas as pl
from jax.experimental.pallas import tpu as pltpu
from jax.experimental.pallas import tpu_sc as plsc

B, S, D, M, V = 64, 50, 128, 2048, 100000
TOPK = 12
TV = 2000
NW = 32
ROWS_PER_W = B // NW
CHUNK = 128
LANES = 16


def _ivec_body(dh_ref, h_ref, w1_ref, b1_ref, w2_ref, b2_ref, out_ref):
    x = dh_ref[...]
    h = h_ref[0]
    hb = jnp.broadcast_to(h[:, None, :], (B, S, D))
    cat = jnp.concatenate([hb, x], axis=2).reshape(B * S, 2 * D)
    t = jnp.tanh(jnp.dot(cat, w1_ref[...],
                         preferred_element_type=jnp.float32) + b1_ref[...])
    q = (jnp.dot(t, w2_ref[...],
                 preferred_element_type=jnp.float32) + b2_ref[...])
    q = q.reshape(B, S, D)
    q = q - jnp.max(q, axis=1, keepdims=True)
    e = jnp.exp(q)
    q = e / jnp.sum(e, axis=1, keepdims=True)
    out_ref[...] = jnp.sum(q * x, axis=1)


def _scores_body(clo_ref, chi_ref, iv_ref, out_ref):
    iv = iv_ref[...]
    out_ref[:, :B] = lax.dot_general(clo_ref[...], iv, (((1,), (1,)), ((), ())),
                                     preferred_element_type=jnp.float32)
    out_ref[:, B:] = lax.dot_general(chi_ref[...], iv, (((1,), (1,)), ((), ())),
                                     preferred_element_type=jnp.float32)


def _flat_index(v, b):
    return jnp.where(v < V // 2, v * (2 * B) + b,
                     (v - V // 2) * (2 * B) + (B + b))


def _topk_body(l_ref, out_ref):
    l = l_ref[...]
    pos = lax.broadcasted_iota(jnp.int32, (B, M), 1)
    cols = []
    for _ in range(TOPK):
        v = jnp.max(l, axis=1, keepdims=True)
        idx = jnp.min(jnp.where(l == v, pos, M), axis=1, keepdims=True)
        cols.append(idx)
        l = jnp.where(pos == idx, -jnp.inf, l)
    out_ref[...] = jnp.concatenate(cols, axis=1)


def _sc_logits(scores_flat, story, gp, kb_len, ctx_len):
    mesh = plsc.VectorSubcoreMesh(core_axis_name="c", subcore_axis_name="s")

    @functools.partial(
        pl.kernel, mesh=mesh,
        out_type=jax.ShapeDtypeStruct((B, M), jnp.float32),
        scratch_types=[
            pltpu.VMEM((M,), jnp.int32),
            pltpu.VMEM((M,), jnp.int32),
            pltpu.VMEM((M,), jnp.float32),
            pltpu.VMEM((M,), jnp.float32),
            pltpu.VMEM((B, LANES), jnp.int32),
            pltpu.VMEM((B, LANES), jnp.int32),
            pltpu.SemaphoreType.DMA,
        ],
    )
    def k(scores_hbm, story_hbm, gp_hbm, kb_hbm, ctx_hbm, out_hbm,
          story_v, idx_v, sc_v, gpv, kb_v, ctx_v, sem):
        cid = lax.axis_index("c")
        sid = lax.axis_index("s")
        w = sid * 2 + cid
        pltpu.sync_copy(kb_hbm, kb_v)
        pltpu.sync_copy(ctx_hbm, ctx_v)
        for r in range(ROWS_PER_W):
            b = w * ROWS_PER_W + r
            pltpu.sync_copy(story_hbm.at[b], story_v)
            pltpu.sync_copy(gp_hbm.at[b], gpv)

            def build(j, carry):
                s16 = story_v[pl.ds(j * LANES, LANES)]
                idx_v[pl.ds(j * LANES, LANES)] = _flat_index(s16, b)
                return carry
            lax.fori_loop(0, M // LANES, build, 0)

            copies = [
                pltpu.async_copy(
                    scores_hbm.at[idx_v.at[pl.ds(j * CHUNK, CHUNK)]],
                    sc_v.at[pl.ds(j * CHUNK, CHUNK)], sem)
                for j in range(M // CHUNK)
            ]
            for cp in copies:
                cp.wait()

            kb16 = kb_v[b]
            ctx16 = ctx_v[b]


            def comp(j, carry):
                posv = j * LANES + lax.iota(jnp.int32, LANES)
                sraw = sc_v[pl.ds(j * LANES, LANES)] * gpv[pl.ds(j * LANES, LANES)]
                badm = ((posv >= kb16) & (posv < ctx16 - 1)) | (posv >= ctx16)
                xm = jnp.where(badm, jnp.float32(-1e9), sraw)
                gpv[pl.ds(j * LANES, LANES)] = 1.0 / (1.0 + jnp.exp(-xm))
                return carry
            lax.fori_loop(0, M // LANES, comp, 0)
            pltpu.sync_copy(gpv, out_hbm.at[b])

    return k(scores_flat, story, gp, kb_len, ctx_len)


def kernel(dh_outputs, dh_hidden, global_pointer, batch_size, story, domain,
           context_len, kb_len, conv_len, memory_mask, memory_story,
           W1, b1, W2, b2, C_know):
    i_vec = pl.pallas_call(
        _ivec_body,
        out_shape=jax.ShapeDtypeStruct((B, D), jnp.float32),
    )(dh_outputs, dh_hidden, W1, b1.reshape(1, D), W2, b2.reshape(1, D))

    scores2d = pl.pallas_call(
        _scores_body,
        grid=(V // (2 * TV),),
        in_specs=[pl.BlockSpec((TV, D), lambda i: (i, 0)),
                  pl.BlockSpec((TV, D), lambda i: (i + V // (2 * TV), 0)),
                  pl.BlockSpec((B, D), lambda i: (0, 0))],
        out_specs=pl.BlockSpec((TV, 2 * B), lambda i: (i, 0)),
        out_shape=jax.ShapeDtypeStruct((V // 2, 2 * B), jnp.float32),
    )(C_know, C_know, i_vec)
    scores_flat = scores2d.reshape(V * B)

    kb_b = jnp.broadcast_to(kb_len.astype(jnp.int32)[:, None], (B, LANES))
    ctx_b = jnp.broadcast_to(context_len.astype(jnp.int32)[:, None], (B, LANES))
    logits = _sc_logits(scores_flat, story, global_pointer, kb_b, ctx_b)

    toppi = pl.pallas_call(
        _topk_body,
        out_shape=jax.ShapeDtypeStruct((B, TOPK), jnp.int32),
    )(logits)
    return toppi, i_vec


# speedup vs baseline: 1.8465x; 1.1120x over previous
import functools

import jax
import jax.numpy as jnp
from jax import lax
from jax.experimental import pallas as pl
from jax.experimental.pallas import tpu as pltpu
from jax.experimental.pallas import tpu_sc as plsc

B, S, D, M, V = 64, 50, 128, 2048, 100000
TOPK = 12
TV = 10000
NW = 32
ROWS_PER_W = B // NW
CHUNK = 128
LANES = 16


def _ivec_body(dh_ref, h_ref, w1_ref, b1_ref, w2_ref, b2_ref, out_ref):
    x = dh_ref[...]
    h = h_ref[0]
    hb = jnp.broadcast_to(h[:, None, :], (B, S, D))
    cat = jnp.concatenate([hb, x], axis=2).reshape(B * S, 2 * D)
    t = jnp.tanh(jnp.dot(cat, w1_ref[...],
                         preferred_element_type=jnp.float32) + b1_ref[...])
    q = (jnp.dot(t, w2_ref[...],
                 preferred_element_type=jnp.float32) + b2_ref[...])
    q = q.reshape(B, S, D)
    q = q - jnp.max(q, axis=1, keepdims=True)
    e = jnp.exp(q)
    q = e / jnp.sum(e, axis=1, keepdims=True)
    out_ref[...] = jnp.sum(q * x, axis=1)


def _scores_body(clo_ref, chi_ref, iv_ref, out_ref):
    iv = iv_ref[...]
    out_ref[:, :B] = lax.dot_general(clo_ref[...], iv, (((1,), (1,)), ((), ())),
                                     preferred_element_type=jnp.float32)
    out_ref[:, B:] = lax.dot_general(chi_ref[...], iv, (((1,), (1,)), ((), ())),
                                     preferred_element_type=jnp.float32)


def _flat_index(v, b):
    return jnp.where(v < V // 2, v * (2 * B) + b,
                     (v - V // 2) * (2 * B) + (B + b))


def _topk_body(l_ref, out_ref):
    l = l_ref[...]
    pos = lax.broadcasted_iota(jnp.int32, (B, M), 1)
    cols = []
    for _ in range(TOPK):
        v = jnp.max(l, axis=1, keepdims=True)
        idx = jnp.min(jnp.where(l == v, pos, M), axis=1, keepdims=True)
        cols.append(idx)
        l = jnp.where(pos == idx, -jnp.inf, l)
    out_ref[...] = jnp.concatenate(cols, axis=1)


def _sc_logits(scores_flat, story, gp, kb_len, ctx_len):
    mesh = plsc.VectorSubcoreMesh(core_axis_name="c", subcore_axis_name="s")

    @functools.partial(
        pl.kernel, mesh=mesh,
        out_type=jax.ShapeDtypeStruct((B, M), jnp.float32),
        scratch_types=[
            pltpu.VMEM((M,), jnp.int32),
            pltpu.VMEM((M,), jnp.int32),
            pltpu.VMEM((M,), jnp.float32),
            pltpu.VMEM((M,), jnp.float32),
            pltpu.VMEM((B, LANES), jnp.int32),
            pltpu.VMEM((B, LANES), jnp.int32),
            pltpu.SemaphoreType.DMA,
        ],
    )
    def k(scores_hbm, story_hbm, gp_hbm, kb_hbm, ctx_hbm, out_hbm,
          story_v, idx_v, sc_v, gpv, kb_v, ctx_v, sem):
        cid = lax.axis_index("c")
        sid = lax.axis_index("s")
        w = sid * 2 + cid
        pltpu.sync_copy(kb_hbm, kb_v)
        pltpu.sync_copy(ctx_hbm, ctx_v)
        for r in range(ROWS_PER_W):
            b = w * ROWS_PER_W + r
            pltpu.sync_copy(story_hbm.at[b], story_v)
            pltpu.sync_copy(gp_hbm.at[b], gpv)

            def build(j, carry):
                s16 = story_v[pl.ds(j * LANES, LANES)]
                idx_v[pl.ds(j * LANES, LANES)] = _flat_index(s16, b)
                return carry
            lax.fori_loop(0, M // LANES, build, 0)

            copies = [
                pltpu.async_copy(
                    scores_hbm.at[idx_v.at[pl.ds(j * CHUNK, CHUNK)]],
                    sc_v.at[pl.ds(j * CHUNK, CHUNK)], sem)
                for j in range(M // CHUNK)
            ]
            for cp in copies:
                cp.wait()

            kb16 = kb_v[b]
            ctx16 = ctx_v[b]


            def comp(j, carry):
                posv = j * LANES + lax.iota(jnp.int32, LANES)
                sraw = sc_v[pl.ds(j * LANES, LANES)] * gpv[pl.ds(j * LANES, LANES)]
                badm = ((posv >= kb16) & (posv < ctx16 - 1)) | (posv >= ctx16)
                xm = jnp.where(badm, jnp.float32(-1e9), sraw)
                gpv[pl.ds(j * LANES, LANES)] = 1.0 / (1.0 + jnp.exp(-xm))
                return carry
            lax.fori_loop(0, M // LANES, comp, 0)
            pltpu.sync_copy(gpv, out_hbm.at[b])

    return k(scores_flat, story, gp, kb_len, ctx_len)


def kernel(dh_outputs, dh_hidden, global_pointer, batch_size, story, domain,
           context_len, kb_len, conv_len, memory_mask, memory_story,
           W1, b1, W2, b2, C_know):
    i_vec = pl.pallas_call(
        _ivec_body,
        out_shape=jax.ShapeDtypeStruct((B, D), jnp.float32),
    )(dh_outputs, dh_hidden, W1, b1.reshape(1, D), W2, b2.reshape(1, D))

    scores2d = pl.pallas_call(
        _scores_body,
        grid=(V // (2 * TV),),
        in_specs=[pl.BlockSpec((TV, D), lambda i: (i, 0)),
                  pl.BlockSpec((TV, D), lambda i: (i + V // (2 * TV), 0)),
                  pl.BlockSpec((B, D), lambda i: (0, 0))],
        out_specs=pl.BlockSpec((TV, 2 * B), lambda i: (i, 0)),
        out_shape=jax.ShapeDtypeStruct((V // 2, 2 * B), jnp.float32),
    )(C_know, C_know, i_vec)
    scores_flat = scores2d.reshape(V * B)

    kb_b = jnp.broadcast_to(kb_len.astype(jnp.int32)[:, None], (B, LANES))
    ctx_b = jnp.broadcast_to(context_len.astype(jnp.int32)[:, None], (B, LANES))
    logits = _sc_logits(scores_flat, story, global_pointer, kb_b, ctx_b)

    toppi = pl.pallas_call(
        _topk_body,
        out_shape=jax.ShapeDtypeStruct((B, TOPK), jnp.int32),
    )(logits)
    return toppi, i_vec
